# baseline (speedup 1.0000x reference)
import jax
import jax.numpy as jnp
from jax.experimental import pallas as pl
from jax.experimental.pallas import tpu as pltpu

_B = 256
_N = 4096
_D = 256
_M = 32
_NT = _D // _M
_NBUF = 2


def _opd_kernel(pos_ref, obs_ref, poly_ref, graph_ref, ap_ref, ag_ref,
                sp_ref, sg_ref, out_ref, c_ref, s_t_ref, obuf_ref, sems):
    i = pl.program_id(0)
    pf = ap_ref.shape[1]

    @pl.when(i == 0)
    def _stage_a():
        pos_t = jnp.transpose(pos_ref[...])
        px = pos_t[0:1, :]
        py = pos_t[1:2, :]
        ox = obs_ref[:, 0:1]
        oy = obs_ref[:, 1:2]
        d = (ox - px) ** 2 + (oy - py) ** 2
        md = jnp.min(d, axis=0, keepdims=True)
        iota = jax.lax.broadcasted_iota(jnp.int32, (_N, _B), 0)
        idx = jnp.min(jnp.where(d == md, iota, _N), axis=0, keepdims=True)
        onehot_t = (iota == idx).astype(jnp.float32)
        cdims = (((0,), (0,)), ((), ()))
        gp = jax.lax.dot_general(onehot_t, poly_ref[...], cdims,
                                 preferred_element_type=jnp.float32)
        gg = jax.lax.dot_general(onehot_t, graph_ref[...], cdims,
                                 preferred_element_type=jnp.float32)
        cp = jnp.dot(gp, ap_ref[...], preferred_element_type=jnp.float32)
        cg = jnp.dot(gg, ag_ref[...], preferred_element_type=jnp.float32)
        c_ref[...] = jnp.concatenate([cp, cg], axis=1)
        s_t_ref[:, 0:pf, :] = jnp.transpose(sp_ref[...], (1, 0, 2))
        s_t_ref[:, pf:, :] = jnp.transpose(sg_ref[...], (1, 0, 2))

    slot = jax.lax.rem(i, _NBUF)

    def _copy(step, buf, m):
        return pltpu.make_async_copy(
            obuf_ref.at[buf, m], out_ref.at[:, step * _M + m, :],
            sems.at[buf])

    @pl.when(i >= _NBUF)
    def _wait_prev():
        for m in range(_M):
            _copy(i - _NBUF, slot, m).wait()

    c = c_ref[...]
    for m in range(_M):
        r = jnp.dot(c, s_t_ref[i * _M + m],
                    preferred_element_type=jnp.float32)
        obuf_ref[slot, m] = r
        _copy(i, slot, m).start()

    @pl.when(i == _NT - 1)
    def _drain():
        for back in range(_NBUF - 1, -1, -1):
            for m in range(_M):
                _copy(i - back, jax.lax.rem(i - back, _NBUF), m).wait()


def kernel(positions, obs_pos, poly_dic, graph_dic, S_poly, S_graph,
           alpha_poly, alpha_graph):
    pf = alpha_poly.shape[1]
    gf = alpha_graph.shape[1]
    k = pf + gf

    opd_maps = pl.pallas_call(
        _opd_kernel,
        grid=(_NT,),
        in_specs=[
            pl.BlockSpec((_B, 2), lambda i: (0, 0)),
            pl.BlockSpec((_N, 2), lambda i: (0, 0)),
            pl.BlockSpec(poly_dic.shape, lambda i: (0, 0)),
            pl.BlockSpec(graph_dic.shape, lambda i: (0, 0)),
            pl.BlockSpec(alpha_poly.shape, lambda i: (0, 0)),
            pl.BlockSpec(alpha_graph.shape, lambda i: (0, 0)),
            pl.BlockSpec((pf, _D, _D), lambda i: (0, 0, 0)),
            pl.BlockSpec((gf, _D, _D), lambda i: (0, 0, 0)),
        ],
        out_specs=pl.BlockSpec(memory_space=pl.ANY),
        out_shape=jax.ShapeDtypeStruct((_B, _D, _D), jnp.float32),
        scratch_shapes=[
            pltpu.VMEM((_B, k), jnp.float32),
            pltpu.VMEM((_D, k, _D), jnp.float32),
            pltpu.VMEM((_NBUF, _M, _B, _D), jnp.float32),
            pltpu.SemaphoreType.DMA((_NBUF,)),
        ],
    )(positions, obs_pos, poly_dic, graph_dic, alpha_poly, alpha_graph,
      S_poly, S_graph)

    return (opd_maps, alpha_graph)


# speedup vs baseline: 1.6616x; 1.6616x over previous
import jax
import jax.numpy as jnp
from jax.experimental import pallas as pl
from jax.experimental.pallas import tpu as pltpu

_B = 256
_N = 4096
_D = 256
_M = 32
_NT = _D // _M
_NBUF = 2


def _opd_kernel(pos_ref, ox_ref, oy_ref, poly_ref, graph_ref, ap_ref, ag_ref,
                sp_ref, sg_ref, out_ref, c_ref, s_all_ref, obuf_ref, sems):
    i = pl.program_id(0)
    pf = ap_ref.shape[1]

    @pl.when(i == 0)
    def _stage_a():
        px = pos_ref[:, 0:1]
        py = pos_ref[:, 1:2]
        ox = ox_ref[...]
        oy = oy_ref[...]
        d = (px - ox) ** 2 + (py - oy) ** 2
        md = jnp.min(d, axis=1, keepdims=True)
        iota = jax.lax.broadcasted_iota(jnp.int32, (_B, _N), 1)
        idx = jnp.min(jnp.where(d == md, iota, _N), axis=1, keepdims=True)
        onehot = (iota == idx).astype(jnp.float32)
        gp = jnp.dot(onehot, poly_ref[...], preferred_element_type=jnp.float32)
        gg = jnp.dot(onehot, graph_ref[...], preferred_element_type=jnp.float32)
        cp = jnp.dot(gp, ap_ref[...], preferred_element_type=jnp.float32)
        cg = jnp.dot(gg, ag_ref[...], preferred_element_type=jnp.float32)
        c_ref[...] = jnp.concatenate([cp, cg], axis=1)
        s_all_ref[0:pf] = sp_ref[...]
        s_all_ref[pf:] = sg_ref[...]

    slot = jax.lax.rem(i, _NBUF)

    def _copy(step, buf):
        return pltpu.make_async_copy(
            obuf_ref.at[buf], out_ref.at[:, pl.ds(step * _M, _M), :],
            sems.at[buf])

    @pl.when(i >= _NBUF)
    def _wait_prev():
        _copy(i - _NBUF, slot).wait()

    s2 = s_all_ref[:, pl.ds(i * _M, _M), :].reshape(2 * pf, _M * _D)
    obuf_ref[slot] = jnp.dot(
        c_ref[...], s2, preferred_element_type=jnp.float32
    ).reshape(_B, _M, _D)
    _copy(i, slot).start()

    @pl.when(i == _NT - 1)
    def _drain():
        for back in range(_NBUF - 1, -1, -1):
            _copy(i - back, jax.lax.rem(i - back, _NBUF)).wait()


def kernel(positions, obs_pos, poly_dic, graph_dic, S_poly, S_graph,
           alpha_poly, alpha_graph):
    pf = alpha_poly.shape[1]
    gf = alpha_graph.shape[1]
    k = pf + gf
    ox = obs_pos[:, 0][None, :]
    oy = obs_pos[:, 1][None, :]

    opd_maps = pl.pallas_call(
        _opd_kernel,
        grid=(_NT,),
        in_specs=[
            pl.BlockSpec((_B, 2), lambda i: (0, 0)),
            pl.BlockSpec((1, _N), lambda i: (0, 0)),
            pl.BlockSpec((1, _N), lambda i: (0, 0)),
            pl.BlockSpec(poly_dic.shape, lambda i: (0, 0)),
            pl.BlockSpec(graph_dic.shape, lambda i: (0, 0)),
            pl.BlockSpec(alpha_poly.shape, lambda i: (0, 0)),
            pl.BlockSpec(alpha_graph.shape, lambda i: (0, 0)),
            pl.BlockSpec((pf, _D, _D), lambda i: (0, 0, 0)),
            pl.BlockSpec((gf, _D, _D), lambda i: (0, 0, 0)),
        ],
        out_specs=pl.BlockSpec(memory_space=pl.ANY),
        out_shape=jax.ShapeDtypeStruct((_B, _D, _D), jnp.float32),
        scratch_shapes=[
            pltpu.VMEM((_B, k), jnp.float32),
            pltpu.VMEM((k, _D, _D), jnp.float32),
            pltpu.VMEM((_NBUF, _B, _M, _D), jnp.float32),
            pltpu.SemaphoreType.DMA((_NBUF,)),
        ],
    )(positions, ox, oy, poly_dic, graph_dic, alpha_poly, alpha_graph,
      S_poly, S_graph)

    return (opd_maps, alpha_graph)


# speedup vs baseline: 1.7234x; 1.0372x over previous
import jax
import jax.numpy as jnp
from jax.experimental import pallas as pl
from jax.experimental.pallas import tpu as pltpu

_B = 256
_N = 4096
_D = 256
_M = 32
_NT = _D // _M


def _opd_kernel(pos_ref, obs_t_ref, poly_ref, graph_ref, ap_ref, ag_ref,
                sp_ref, sg_ref, out_ref, c_ref, s_all_ref, obuf_ref, sems):
    i = pl.program_id(0)
    pf = ap_ref.shape[1]

    @pl.when(i == 0)
    def _stage_a():
        px = pos_ref[:, 0:1]
        py = pos_ref[:, 1:2]
        ox = obs_t_ref[0:1, :]
        oy = obs_t_ref[1:2, :]
        d = (px - ox) ** 2 + (py - oy) ** 2
        md = jnp.min(d, axis=1, keepdims=True)
        iota = jax.lax.broadcasted_iota(jnp.int32, (_B, _N), 1)
        idx = jnp.min(jnp.where(d == md, iota, _N), axis=1, keepdims=True)
        onehot = (iota == idx).astype(jnp.float32)
        gp = jnp.dot(onehot, poly_ref[...], preferred_element_type=jnp.float32)
        gg = jnp.dot(onehot, graph_ref[...], preferred_element_type=jnp.float32)
        cp = jnp.dot(gp, ap_ref[...], preferred_element_type=jnp.float32)
        cg = jnp.dot(gg, ag_ref[...], preferred_element_type=jnp.float32)
        c_ref[...] = jnp.concatenate([cp, cg], axis=1)
        s_all_ref[0:pf] = sp_ref[...]
        s_all_ref[pf:] = sg_ref[...]

    slot = jax.lax.rem(i, 2)

    @pl.when(i >= 2)
    def _wait_prev():
        pltpu.make_async_copy(
            obuf_ref.at[slot], out_ref.at[:, pl.ds((i - 2) * _M, _M), :],
            sems.at[slot]).wait()

    s2 = s_all_ref[:, pl.ds(i * _M, _M), :].reshape(2 * pf, _M * _D)
    r = jnp.dot(c_ref[...], s2, preferred_element_type=jnp.float32)
    obuf_ref[slot] = r.reshape(_B, _M, _D)
    cp_out = pltpu.make_async_copy(
        obuf_ref.at[slot], out_ref.at[:, pl.ds(i * _M, _M), :], sems.at[slot])
    cp_out.start()

    @pl.when(i == _NT - 1)
    def _drain():
        pltpu.make_async_copy(
            obuf_ref.at[1 - slot],
            out_ref.at[:, pl.ds((i - 1) * _M, _M), :],
            sems.at[1 - slot]).wait()
        cp_out.wait()


def kernel(positions, obs_pos, poly_dic, graph_dic, S_poly, S_graph,
           alpha_poly, alpha_graph):
    pf = alpha_poly.shape[1]
    gf = alpha_graph.shape[1]
    k = pf + gf
    obs_t = obs_pos.T

    opd_maps = pl.pallas_call(
        _opd_kernel,
        grid=(_NT,),
        in_specs=[
            pl.BlockSpec((_B, 2), lambda i: (0, 0)),
            pl.BlockSpec((2, _N), lambda i: (0, 0)),
            pl.BlockSpec(poly_dic.shape, lambda i: (0, 0)),
            pl.BlockSpec(graph_dic.shape, lambda i: (0, 0)),
            pl.BlockSpec(alpha_poly.shape, lambda i: (0, 0)),
            pl.BlockSpec(alpha_graph.shape, lambda i: (0, 0)),
            pl.BlockSpec((pf, _D, _D), lambda i: (0, 0, 0)),
            pl.BlockSpec((gf, _D, _D), lambda i: (0, 0, 0)),
        ],
        out_specs=pl.BlockSpec(memory_space=pl.ANY),
        out_shape=jax.ShapeDtypeStruct((_B, _D, _D), jnp.float32),
        scratch_shapes=[
            pltpu.VMEM((_B, k), jnp.float32),
            pltpu.VMEM((k, _D, _D), jnp.float32),
            pltpu.VMEM((2, _B, _M, _D), jnp.float32),
            pltpu.SemaphoreType.DMA((2,)),
        ],
    )(positions, obs_t, poly_dic, graph_dic, alpha_poly, alpha_graph,
      S_poly, S_graph)

    return (opd_maps, alpha_graph)


# speedup vs baseline: 1.7441x; 1.0120x over previous
import jax
import jax.numpy as jnp
from jax.experimental import pallas as pl
from jax.experimental.pallas import tpu as pltpu

_B = 256
_N = 4096
_D = 256
_M = 32
_NT = _D // _M
_NBUF = 3


def _opd_kernel(pos_ref, obs_t_ref, poly_ref, graph_ref, ap_ref, ag_ref,
                sp_ref, sg_ref, out_ref, c_ref, s_all_ref, obuf_ref, sems):
    i = pl.program_id(0)
    pf = ap_ref.shape[1]

    @pl.when(i == 0)
    def _stage_a():
        px = pos_ref[:, 0:1]
        py = pos_ref[:, 1:2]
        ox = obs_t_ref[0:1, :]
        oy = obs_t_ref[1:2, :]
        d = (px - ox) ** 2 + (py - oy) ** 2
        md = jnp.min(d, axis=1, keepdims=True)
        iota = jax.lax.broadcasted_iota(jnp.int32, (_B, _N), 1)
        idx = jnp.min(jnp.where(d == md, iota, _N), axis=1, keepdims=True)
        onehot = (iota == idx).astype(jnp.float32)
        gp = jnp.dot(onehot, poly_ref[...], preferred_element_type=jnp.float32)
        gg = jnp.dot(onehot, graph_ref[...], preferred_element_type=jnp.float32)
        cp = jnp.dot(gp, ap_ref[...], preferred_element_type=jnp.float32)
        cg = jnp.dot(gg, ag_ref[...], preferred_element_type=jnp.float32)
        c_ref[...] = jnp.concatenate([cp, cg], axis=1)
        s_all_ref[0:pf] = sp_ref[...]
        s_all_ref[pf:] = sg_ref[...]

    slot = jax.lax.rem(i, _NBUF)

    def _copy(step, buf):
        return pltpu.make_async_copy(
            obuf_ref.at[buf], out_ref.at[:, pl.ds(step * _M, _M), :],
            sems.at[buf])

    @pl.when(i >= _NBUF)
    def _wait_prev():
        _copy(i - _NBUF, slot).wait()

    s2 = s_all_ref[:, pl.ds(i * _M, _M), :].reshape(2 * pf, _M * _D)
    r = jnp.dot(c_ref[...], s2, preferred_element_type=jnp.float32)
    obuf_ref[slot] = r.reshape(_B, _M, _D)
    _copy(i, slot).start()

    @pl.when(i == _NT - 1)
    def _drain():
        for back in range(_NBUF - 1, -1, -1):
            _copy(i - back, jax.lax.rem(i - back, _NBUF)).wait()


def kernel(positions, obs_pos, poly_dic, graph_dic, S_poly, S_graph,
           alpha_poly, alpha_graph):
    pf = alpha_poly.shape[1]
    gf = alpha_graph.shape[1]
    k = pf + gf
    obs_t = obs_pos.T

    opd_maps = pl.pallas_call(
        _opd_kernel,
        grid=(_NT,),
        in_specs=[
            pl.BlockSpec((_B, 2), lambda i: (0, 0)),
            pl.BlockSpec((2, _N), lambda i: (0, 0)),
            pl.BlockSpec(poly_dic.shape, lambda i: (0, 0)),
            pl.BlockSpec(graph_dic.shape, lambda i: (0, 0)),
            pl.BlockSpec(alpha_poly.shape, lambda i: (0, 0)),
            pl.BlockSpec(alpha_graph.shape, lambda i: (0, 0)),
            pl.BlockSpec((pf, _D, _D), lambda i: (0, 0, 0)),
            pl.BlockSpec((gf, _D, _D), lambda i: (0, 0, 0)),
        ],
        out_specs=pl.BlockSpec(memory_space=pl.ANY),
        out_shape=jax.ShapeDtypeStruct((_B, _D, _D), jnp.float32),
        scratch_shapes=[
            pltpu.VMEM((_B, k), jnp.float32),
            pltpu.VMEM((k, _D, _D), jnp.float32),
            pltpu.VMEM((_NBUF, _B, _M, _D), jnp.float32),
            pltpu.SemaphoreType.DMA((_NBUF,)),
        ],
    )(positions, obs_t, poly_dic, graph_dic, alpha_poly, alpha_graph,
      S_poly, S_graph)

    return (opd_maps, alpha_graph)
